# baseline (speedup 1.0000x reference)
import jax
import jax.numpy as jnp
from jax.experimental import pallas as pl
from jax.experimental.pallas import tpu as pltpu

N = 2048
IMG_FLAT = 3 * 32 * 32
BI = 512
BR = 256
NI = N // BI
NB = N // BR
S_PA = NI
S_MID = NI + NB
S_DEC = S_MID + 1


def _body(pa0_ref, pa1_ref, pa2_ref, x_ref, wenc_ref, wdec_ref,
          w0, w1, w2, w3, w4, w5, w6, w7,
          out_ref, a_s, hp0f_s, hp0b_s, p1_s, pfin_s):
    j = pl.program_id(0)

    @pl.when(j < S_PA)
    def _encode():
        p0_blk = jnp.tanh(
            jnp.dot(
                x_ref[...].astype(jnp.bfloat16),
                wenc_ref[...].astype(jnp.bfloat16),
                preferred_element_type=jnp.float32,
            )
        )
        hp0_blk = jnp.dot(p0_blk, w0[...], preferred_element_type=jnp.float32)
        hp0f_s[pl.ds(j * BI, BI), :] = hp0_blk
        hp0b_s[pl.ds(j * BI, BI), :] = hp0_blk.astype(jnp.bfloat16)

    @pl.when(jnp.logical_and(j >= S_PA, j < S_MID))
    def _build():
        blk = j - S_PA
        a_blk = (
            (pa0_ref[0] + pa1_ref[0] + pa2_ref[0]) * (1.0 / 3.0)
        ).astype(jnp.bfloat16)
        a_s[pl.ds(blk * BR, BR), :] = a_blk
        agg1 = jnp.dot(a_blk, hp0b_s[...], preferred_element_type=jnp.float32)
        p1_s[pl.ds(blk * BR, BR), :] = jnp.tanh(
            hp0f_s[pl.ds(blk * BR, BR), :] + agg1
        )

    @pl.when(j == S_MID)
    def _layers():
        A = a_s[...]
        p = p1_s[...]
        for w_ref in (w1, w2, w3, w4, w5, w6):
            w = w_ref[...]
            hp = jnp.dot(p, w, preferred_element_type=jnp.float32)
            agg = jnp.dot(
                A, hp.astype(jnp.bfloat16), preferred_element_type=jnp.float32
            )
            p = jnp.tanh(hp + agg)
        pfin_s[...] = jnp.dot(p, w7[...], preferred_element_type=jnp.float32)

    @pl.when(j >= S_DEC)
    def _decode():
        blk = j - S_DEC
        agg8 = jnp.dot(
            a_s[pl.ds(blk * BR, BR), :],
            pfin_s[...].astype(jnp.bfloat16),
            preferred_element_type=jnp.float32,
        )
        p_blk = jnp.tanh(pfin_s[pl.ds(blk * BR, BR), :] + agg8)
        out_ref[...] = jnp.dot(
            p_blk.astype(jnp.bfloat16),
            wdec_ref[...].astype(jnp.bfloat16),
            preferred_element_type=jnp.float32,
        )


def _clamp(lo, x, hi):
    return jnp.minimum(jnp.maximum(x, lo), hi)


def kernel(imgs, primal_adjacency_tensor, dual_adjacency_tensor, dual_nodes, params):
    del dual_adjacency_tensor, dual_nodes
    n = imgs.shape[0]
    imgs_flat = imgs.reshape(n, IMG_FLAT)

    ws = [params["Wp_enc_%d" % i] for i in range(4)] + [
        params["Wp_dec_%d" % i] for i in range(4)
    ]

    recon_call = pl.pallas_call(
        _body,
        grid=(S_DEC + NB,),
        in_specs=[
            pl.BlockSpec((1, BR, N), lambda j: (0, _clamp(0, j - S_PA, NB - 1), 0)),
            pl.BlockSpec((1, BR, N), lambda j: (1, _clamp(0, j - S_PA, NB - 1), 0)),
            pl.BlockSpec((1, BR, N), lambda j: (2, _clamp(0, j - S_PA, NB - 1), 0)),
            pl.BlockSpec((BI, IMG_FLAT), lambda j: (jnp.minimum(j, NI - 1), 0)),
            pl.BlockSpec((IMG_FLAT, 64), lambda j: (0, 0)),
            pl.BlockSpec((64, IMG_FLAT), lambda j: (0, 0)),
        ]
        + [pl.BlockSpec(w.shape, lambda j: (0, 0)) for w in ws],
        out_specs=pl.BlockSpec(
            (BR, IMG_FLAT), lambda j: (_clamp(0, j - S_DEC, NB - 1), 0)
        ),
        out_shape=jax.ShapeDtypeStruct((n, IMG_FLAT), jnp.float32),
        scratch_shapes=[
            pltpu.VMEM((N, N), jnp.bfloat16),
            pltpu.VMEM((N, 32), jnp.float32),
            pltpu.VMEM((N, 32), jnp.bfloat16),
            pltpu.VMEM((N, 32), jnp.float32),
            pltpu.VMEM((N, 64), jnp.float32),
        ],
    )
    pa = primal_adjacency_tensor
    recon = recon_call(pa, pa, pa, imgs_flat,
                       params["W_img_enc"], params["W_img_dec"], *ws)

    return recon.reshape(imgs.shape)
